# Initial kernel scaffold; baseline (speedup 1.0000x reference)
import jax
import jax.numpy as jnp
from jax.experimental import pallas as pl


def kernel(hidden_states, W_gate, bias_corr, Wg, Wu, Wd, Ws_g, Ws_u, Ws_d):
    raise NotImplementedError("write your pallas kernel here")


# speedup vs baseline: 4.2534x; 4.2534x over previous
import functools

import jax
import jax.numpy as jnp
from jax.experimental import pallas as pl
from jax.experimental.pallas import tpu as pltpu

H = 1024
E = 64
I = 256
G = 8
TKG = 4
K = 2
IS = 512
T = 2048
EPG = E // G


def _routing_dense(x, w_gate, bias):
    logits = jnp.dot(x, w_gate, preferred_element_type=jnp.float32)
    scores = jax.nn.sigmoid(logits) + bias
    gs = jnp.concatenate(
        [jnp.max(scores[:, g * EPG:(g + 1) * EPG], axis=1, keepdims=True)
         for g in range(G)], axis=1)
    giota = jax.lax.broadcasted_iota(jnp.int32, (T, G), 1)
    gmask = jnp.zeros((T, G), jnp.float32)
    cur = gs
    for _ in range(TKG):
        m = jnp.max(cur, axis=1, keepdims=True)
        sel_idx = jnp.min(jnp.where(cur == m, giota, G), axis=1, keepdims=True)
        sel = giota == sel_idx
        gmask = gmask + sel.astype(jnp.float32)
        cur = jnp.where(sel, -jnp.inf, cur)
    emask = jnp.concatenate(
        [jnp.broadcast_to(gmask[:, g:g + 1], (T, EPG)) for g in range(G)],
        axis=1)
    masked = scores * emask
    eiota = jax.lax.broadcasted_iota(jnp.int32, (T, E), 1)
    cur = masked
    ws, sels = [], []
    for _ in range(K):
        m = jnp.max(cur, axis=1, keepdims=True)
        si = jnp.min(jnp.where(cur == m, eiota, E), axis=1, keepdims=True)
        sel = eiota == si
        ws.append(m)
        sels.append(sel)
        cur = jnp.where(sel, -jnp.inf, cur)
    denom = ws[0] + ws[1] + 1e-8
    wd = (ws[0] / denom) * sels[0].astype(jnp.float32)
    wd = wd + (ws[1] / denom) * sels[1].astype(jnp.float32)
    return wd


def _gate_shared_kernel(x_ref, wgate_ref, bias_ref, wsg_ref, wsu_ref, wsd_ref,
                        wdense_ref, shared_ref):
    x = x_ref[...]
    wdense_ref[...] = _routing_dense(x, wgate_ref[...], bias_ref[...])
    g = jnp.dot(x, wsg_ref[...], preferred_element_type=jnp.float32)
    u = jnp.dot(x, wsu_ref[...], preferred_element_type=jnp.float32)
    h = jax.nn.silu(g) * u
    shared_ref[...] = jnp.dot(h, wsd_ref[...], preferred_element_type=jnp.float32)


def _dense_moe_kernel(wdense_ref, x_ref, wg_ref, wu_ref, wd_ref, shared_ref,
                      out_ref):
    e = pl.program_id(0)
    x = x_ref[...]
    g = jnp.dot(x, wg_ref[0], preferred_element_type=jnp.float32)
    u = jnp.dot(x, wu_ref[0], preferred_element_type=jnp.float32)
    h = jax.nn.silu(g) * u
    y = jnp.dot(h, wd_ref[0], preferred_element_type=jnp.float32)
    eiota = jax.lax.broadcasted_iota(jnp.int32, (T, E), 1)
    wcol = jnp.sum(wdense_ref[...] * (eiota == e).astype(jnp.float32),
                   axis=1, keepdims=True)

    @pl.when(e == 0)
    def _init():
        out_ref[...] = shared_ref[...]

    out_ref[...] += y * wcol


def kernel(hidden_states, W_gate, bias_corr, Wg, Wu, Wd, Ws_g, Ws_u, Ws_d):
    x = hidden_states.reshape(T, H)
    bias2d = bias_corr.reshape(1, E)

    wdense, shared = pl.pallas_call(
        _gate_shared_kernel,
        out_shape=(
            jax.ShapeDtypeStruct((T, E), jnp.float32),
            jax.ShapeDtypeStruct((T, H), jnp.float32),
        ),
    )(x, W_gate, bias2d, Ws_g, Ws_u, Ws_d)

    out = pl.pallas_call(
        _dense_moe_kernel,
        grid=(E,),
        in_specs=[
            pl.BlockSpec((T, E), lambda e: (0, 0)),
            pl.BlockSpec((T, H), lambda e: (0, 0)),
            pl.BlockSpec((1, H, I), lambda e: (e, 0, 0)),
            pl.BlockSpec((1, H, I), lambda e: (e, 0, 0)),
            pl.BlockSpec((1, I, H), lambda e: (e, 0, 0)),
            pl.BlockSpec((T, H), lambda e: (0, 0)),
        ],
        out_specs=pl.BlockSpec((T, H), lambda e: (0, 0)),
        out_shape=jax.ShapeDtypeStruct((T, H), jnp.float32),
    )(wdense, x, Wg, Wu, Wd, shared)

    return out.reshape(1, T, H)
